# baseline (speedup 1.0000x reference)
import functools

import jax
import jax.numpy as jnp
from jax import lax
from jax.experimental import pallas as pl
from jax.experimental.pallas import tpu as pltpu
from jax.experimental.pallas import tpu_sc as plsc

N_NODES = 10000
N_EDGES = 320000
D = 128
N_CLASS = 10

NC = 2
NS = 16
NW = NC * NS

NPAD = 10240
ROWS_PER_SUB = NPAD // NS
CHUNK = 128
CH_PER_W = 80
EROWS = NW * CH_PER_W
EPAD = EROWS * CHUNK
DH = D // 2
G = 8

_MESH = dict(core_axis_name="c", subcore_axis_name="s")


def _sc_degrees(edges3, ones8, zeros8):
    @functools.partial(
        pl.kernel,
        out_type=jax.ShapeDtypeStruct((NC, 2, NPAD, 8), jnp.float32),
        mesh=plsc.VectorSubcoreMesh(**_MESH),
        compiler_params=pltpu.CompilerParams(use_tc_tiling_on_sc=False),
        scratch_types=(
            [pltpu.VMEM((2, CHUNK), jnp.int32) for _ in range(G)]
            + [
                pltpu.VMEM((CHUNK, 8), jnp.float32),
                pltpu.VMEM((ROWS_PER_SUB, 8), jnp.float32),
                pltpu.VMEM_SHARED((NPAD, 8), jnp.float32),
                pltpu.VMEM_SHARED((NPAD, 8), jnp.float32),
                pltpu.SemaphoreType.DMA,
                pltpu.SemaphoreType.DMA,
            ]
        ),
    )
    def deg_kernel(edg_h, ones_h, zer_h, out_h, *scr):
        idx = scr[0:G]
        ones_v, zer_v, acc_s, acc_d, sem_i, sem_p = scr[G:]
        c = lax.axis_index("c")
        s = lax.axis_index("s")
        wid = s * NC + c
        pltpu.sync_copy(zer_h, zer_v)
        pltpu.sync_copy(zer_v, acc_s.at[pl.ds(s * ROWS_PER_SUB, ROWS_PER_SUB)])
        pltpu.sync_copy(zer_v, acc_d.at[pl.ds(s * ROWS_PER_SUB, ROWS_PER_SUB)])
        pltpu.sync_copy(ones_h, ones_v)
        plsc.subcore_barrier()

        def body(i, carry):
            base = wid * CH_PER_W + i * G
            loads = [
                pltpu.async_copy(edg_h.at[base + j], idx[j], sem_i)
                for j in range(G)
            ]
            for d in loads:
                d.wait()
            puts = []
            for j in range(G):
                puts.append(
                    pltpu.async_copy(ones_v, acc_s.at[idx[j].at[0]], sem_p,
                                     add=True))
                puts.append(
                    pltpu.async_copy(ones_v, acc_d.at[idx[j].at[1]], sem_p,
                                     add=True))
            for d in puts:
                d.wait()
            return carry

        lax.fori_loop(0, CH_PER_W // G, body, 0)
        plsc.subcore_barrier()
        sl = pl.ds(s * ROWS_PER_SUB, ROWS_PER_SUB)
        pltpu.sync_copy(acc_s.at[sl], out_h.at[c, 0, sl])
        pltpu.sync_copy(acc_d.at[sl], out_h.at[c, 1, sl])

    return deg_kernel(edges3, ones8, zeros8)


RPS = EROWS // NS


def _sc_scatter(h, edges3, zerosD):
    @functools.partial(
        pl.kernel,
        out_type=jax.ShapeDtypeStruct((2, NPAD, DH), jnp.float32),
        mesh=plsc.VectorSubcoreMesh(**_MESH),
        compiler_params=pltpu.CompilerParams(use_tc_tiling_on_sc=False),
        scratch_types=(
            [pltpu.VMEM((2, CHUNK), jnp.int32) for _ in range(G)]
            + [pltpu.VMEM((CHUNK, DH), jnp.float32) for _ in range(G)]
            + [
                pltpu.VMEM((CHUNK, DH), jnp.float32),
                pltpu.VMEM_SHARED((NPAD, DH), jnp.float32),
                pltpu.SemaphoreType.DMA,
                pltpu.SemaphoreType.DMA,
            ]
            + [pltpu.SemaphoreType.DMA for _ in range(G)]
        ),
    )
    def mp_kernel(h_h, edg_h, zer_h, out_h, *scr):
        idx = scr[0:G]
        rows = scr[G:2 * G]
        zer_v = scr[2 * G]
        acc = scr[2 * G + 1]
        sem_i = scr[2 * G + 2]
        sem_p = scr[2 * G + 3]
        sem_g = scr[2 * G + 4:2 * G + 4 + G]
        c = lax.axis_index("c")
        s = lax.axis_index("s")
        pltpu.sync_copy(zer_h, zer_v)
        for k in range(ROWS_PER_SUB // CHUNK):
            pltpu.sync_copy(
                zer_v, acc.at[pl.ds(s * ROWS_PER_SUB + k * CHUNK, CHUNK)])
        plsc.subcore_barrier()

        def run(table):
            def body(i, carry):
                base = s * RPS + i * G
                loads = [
                    pltpu.async_copy(edg_h.at[base + j], idx[j], sem_i)
                    for j in range(G)
                ]
                for d in loads:
                    d.wait()
                gets = [
                    pltpu.async_copy(table.at[idx[j].at[0]], rows[j], sem_g[j])
                    for j in range(G)
                ]
                puts = []
                for j in range(G):
                    gets[j].wait()
                    puts.append(
                        pltpu.async_copy(
                            rows[j], acc.at[idx[j].at[1]], sem_p, add=True))
                for d in puts:
                    d.wait()
                return carry

            lax.fori_loop(0, RPS // G, body, 0)

        @pl.when(c == 0)
        def _():
            run(h_h.at[0])

        @pl.when(c == 1)
        def _():
            run(h_h.at[1])

        plsc.subcore_barrier()
        sl = pl.ds(s * ROWS_PER_SUB, ROWS_PER_SUB)
        pltpu.sync_copy(acc.at[sl], out_h.at[c, sl])

    return mp_kernel(h, edges3, zerosD)


_BLK = 1024
_GRID = NPAD // _BLK


def _norms(degp_blk):
    d_out = degp_blk[0, 0] + degp_blk[1, 0]
    d_in = degp_blk[0, 1] + degp_blk[1, 1]
    n_src = lax.rsqrt(jnp.where(d_out > 0.0, d_out, 1.0))[:, 0:1]
    n_dst = lax.rsqrt(jnp.where(d_in > 0.0, d_in, 1.0))[:, 0:1]
    return n_src, n_dst


def _split(h):
    return jnp.stack([h[:, :DH], h[:, DH:]], axis=0)


def _tc_scale(featsp, degp):
    def body(f_ref, dg_ref, o_ref):
        n_src, _ = _norms(dg_ref[...])
        o_ref[...] = _split(f_ref[...] * n_src)

    return pl.pallas_call(
        body,
        grid=(_GRID,),
        in_specs=[
            pl.BlockSpec((_BLK, D), lambda i: (i, 0)),
            pl.BlockSpec((NC, 2, _BLK, 8), lambda i: (0, 0, i, 0)),
        ],
        out_specs=pl.BlockSpec((2, _BLK, DH), lambda i: (0, i, 0)),
        out_shape=jax.ShapeDtypeStruct((2, NPAD, DH), jnp.float32),
    )(featsp, degp)


def _merge_parts(p):
    return jnp.concatenate([p[0], p[1]], axis=1)


def _tc_layer(part, degp, W, b):
    def body(p_ref, dg_ref, w_ref, b_ref, o_ref):
        i = pl.program_id(0)
        n_src, n_dst = _norms(dg_ref[...])
        agg = _merge_parts(p_ref[...]) * n_dst
        h = jnp.dot(agg, w_ref[...], preferred_element_type=jnp.float32)
        h = jnp.maximum(h + b_ref[...], 0.0)
        rows = i * _BLK + lax.broadcasted_iota(jnp.int32, (_BLK, 1), 0)
        h = jnp.where(rows < N_NODES, h, 0.0)
        o_ref[...] = _split(h * n_src)

    return pl.pallas_call(
        body,
        grid=(_GRID,),
        in_specs=[
            pl.BlockSpec((2, _BLK, DH), lambda i: (0, i, 0)),
            pl.BlockSpec((NC, 2, _BLK, 8), lambda i: (0, 0, i, 0)),
            pl.BlockSpec((D, D), lambda i: (0, 0)),
            pl.BlockSpec((1, D), lambda i: (0, 0)),
        ],
        out_specs=pl.BlockSpec((2, _BLK, DH), lambda i: (0, i, 0)),
        out_shape=jax.ShapeDtypeStruct((2, NPAD, DH), jnp.float32),
    )(part, degp, W, b)


def _tc_layer_sum(part, degp, W, b):
    def body(p_ref, dg_ref, w_ref, b_ref, o_ref):
        i = pl.program_id(0)
        _, n_dst = _norms(dg_ref[...])
        agg = _merge_parts(p_ref[...]) * n_dst
        h = jnp.dot(agg, w_ref[...], preferred_element_type=jnp.float32)
        h = jnp.maximum(h + b_ref[...], 0.0)
        rows = i * _BLK + lax.broadcasted_iota(jnp.int32, (_BLK, 1), 0)
        h = jnp.where(rows < N_NODES, h, 0.0)

        @pl.when(i == 0)
        def _():
            o_ref[...] = jnp.zeros_like(o_ref)

        o_ref[...] += jnp.sum(h, axis=0, keepdims=True)

    return pl.pallas_call(
        body,
        grid=(_GRID,),
        in_specs=[
            pl.BlockSpec((2, _BLK, DH), lambda i: (0, i, 0)),
            pl.BlockSpec((NC, 2, _BLK, 8), lambda i: (0, 0, i, 0)),
            pl.BlockSpec((D, D), lambda i: (0, 0)),
            pl.BlockSpec((1, D), lambda i: (0, 0)),
        ],
        out_specs=pl.BlockSpec((1, D), lambda i: (0, 0)),
        out_shape=jax.ShapeDtypeStruct((1, D), jnp.float32),
    )(part, degp, W, b)


def _tc_head(sums, Wf1, bf1, Wf2, bf2):
    def body(s_ref, w1_ref, b1_ref, w2_ref, b2_ref, o_ref):
        g = s_ref[...] * (1.0 / N_NODES)
        x = jnp.dot(g, w1_ref[...], preferred_element_type=jnp.float32)
        x = jnp.maximum(x + b1_ref[...], 0.0)
        l = jnp.dot(x, w2_ref[...], preferred_element_type=jnp.float32)
        l = l + b2_ref[...]
        m = jnp.max(l, axis=1, keepdims=True)
        e = jnp.exp(l - m)
        o_ref[...] = e / jnp.sum(e, axis=1, keepdims=True)

    return pl.pallas_call(
        body,
        out_shape=jax.ShapeDtypeStruct((1, N_CLASS), jnp.float32),
    )(sums, Wf1, bf1, Wf2, bf2)


def kernel(feats, edge_index, W1, b1, W2, b2, Wf1, bf1, Wf2, bf2):
    src = edge_index[0].astype(jnp.int32)
    dst = edge_index[1].astype(jnp.int32)
    pad = N_NODES + jnp.arange(EPAD - N_EDGES, dtype=jnp.int32) % (
        NPAD - N_NODES)
    src2 = jnp.concatenate([src, pad]).reshape(EROWS, CHUNK)
    dst2 = jnp.concatenate([dst, pad]).reshape(EROWS, CHUNK)
    edges3 = jnp.stack([src2, dst2], axis=1)
    featsp = jnp.pad(feats, ((0, NPAD - N_NODES), (0, 0)))

    ones8 = jnp.ones((CHUNK, 8), jnp.float32)
    zeros8 = jnp.zeros((ROWS_PER_SUB, 8), jnp.float32)
    zerosD = jnp.zeros((CHUNK, DH), jnp.float32)

    degp = _sc_degrees(edges3, ones8, zeros8)
    h0s = _tc_scale(featsp, degp)
    p1 = _sc_scatter(h0s, edges3, zerosD)
    h1s = _tc_layer(p1, degp, W1, b1.reshape(1, D))
    p2 = _sc_scatter(h1s, edges3, zerosD)
    sums = _tc_layer_sum(p2, degp, W2, b2.reshape(1, D))
    return _tc_head(sums, Wf1, bf1.reshape(1, N_CLASS), Wf2,
                    bf2.reshape(1, N_CLASS))


# speedup vs baseline: 1.0015x; 1.0015x over previous
import functools

import jax
import jax.numpy as jnp
from jax import lax
from jax.experimental import pallas as pl
from jax.experimental.pallas import tpu as pltpu
from jax.experimental.pallas import tpu_sc as plsc

N_NODES = 10000
N_EDGES = 320000
D = 128
N_CLASS = 10

NC = 2
NS = 16
NW = NC * NS

NPAD = 10240
ROWS_PER_SUB = NPAD // NS
CHUNK = 128
CH_PER_W = 80
EROWS = NW * CH_PER_W
EPAD = EROWS * CHUNK
DH = D // 2
G = 8

_MESH = dict(core_axis_name="c", subcore_axis_name="s")


def _sc_degrees(edges3, ones8, zeros8):
    @functools.partial(
        pl.kernel,
        out_type=jax.ShapeDtypeStruct((NC, 2, NPAD, 8), jnp.float32),
        mesh=plsc.VectorSubcoreMesh(**_MESH),
        compiler_params=pltpu.CompilerParams(use_tc_tiling_on_sc=False),
        scratch_types=(
            [pltpu.VMEM((2, CHUNK), jnp.int32) for _ in range(G)]
            + [
                pltpu.VMEM((CHUNK, 8), jnp.float32),
                pltpu.VMEM((ROWS_PER_SUB, 8), jnp.float32),
                pltpu.VMEM_SHARED((NPAD, 8), jnp.float32),
                pltpu.VMEM_SHARED((NPAD, 8), jnp.float32),
                pltpu.SemaphoreType.DMA,
                pltpu.SemaphoreType.DMA,
            ]
        ),
    )
    def deg_kernel(edg_h, ones_h, zer_h, out_h, *scr):
        idx = scr[0:G]
        ones_v, zer_v, acc_s, acc_d, sem_i, sem_p = scr[G:]
        c = lax.axis_index("c")
        s = lax.axis_index("s")
        wid = s * NC + c
        pltpu.sync_copy(zer_h, zer_v)
        pltpu.sync_copy(zer_v, acc_s.at[pl.ds(s * ROWS_PER_SUB, ROWS_PER_SUB)])
        pltpu.sync_copy(zer_v, acc_d.at[pl.ds(s * ROWS_PER_SUB, ROWS_PER_SUB)])
        pltpu.sync_copy(ones_h, ones_v)
        plsc.subcore_barrier()

        def body(i, carry):
            base = wid * CH_PER_W + i * G
            loads = [
                pltpu.async_copy(edg_h.at[base + j], idx[j], sem_i)
                for j in range(G)
            ]
            for d in loads:
                d.wait()
            puts = []
            for j in range(G):
                puts.append(
                    pltpu.async_copy(ones_v, acc_s.at[idx[j].at[0]], sem_p,
                                     add=True))
                puts.append(
                    pltpu.async_copy(ones_v, acc_d.at[idx[j].at[1]], sem_p,
                                     add=True))
            for d in puts:
                d.wait()
            return carry

        lax.fori_loop(0, CH_PER_W // G, body, 0)
        plsc.subcore_barrier()
        sl = pl.ds(s * ROWS_PER_SUB, ROWS_PER_SUB)
        pltpu.sync_copy(acc_s.at[sl], out_h.at[c, 0, sl])
        pltpu.sync_copy(acc_d.at[sl], out_h.at[c, 1, sl])

    return deg_kernel(edges3, ones8, zeros8)


RPS = EROWS // NS


def _sc_scatter(h, edges3, zerosD):
    @functools.partial(
        pl.kernel,
        out_type=jax.ShapeDtypeStruct((2, NPAD, DH), jnp.float32),
        mesh=plsc.VectorSubcoreMesh(**_MESH),
        compiler_params=pltpu.CompilerParams(use_tc_tiling_on_sc=False),
        scratch_types=(
            [pltpu.VMEM((2, CHUNK), jnp.int32) for _ in range(G)]
            + [pltpu.VMEM((CHUNK, DH), jnp.float32) for _ in range(G)]
            + [
                pltpu.VMEM((CHUNK, DH), jnp.float32),
                pltpu.VMEM_SHARED((NPAD, DH), jnp.float32),
                pltpu.SemaphoreType.DMA,
                pltpu.SemaphoreType.DMA,
            ]
            + [pltpu.SemaphoreType.DMA for _ in range(G)]
        ),
    )
    def mp_kernel(h_h, edg_h, zer_h, out_h, *scr):
        idx = scr[0:G]
        rows = scr[G:2 * G]
        zer_v = scr[2 * G]
        acc = scr[2 * G + 1]
        sem_i = scr[2 * G + 2]
        sem_p = scr[2 * G + 3]
        sem_g = scr[2 * G + 4:2 * G + 4 + G]
        c = lax.axis_index("c")
        s = lax.axis_index("s")
        pltpu.sync_copy(zer_h, zer_v)
        for k in range(ROWS_PER_SUB // CHUNK):
            pltpu.sync_copy(
                zer_v, acc.at[pl.ds(s * ROWS_PER_SUB + k * CHUNK, CHUNK)])
        plsc.subcore_barrier()

        def run(table):
            def body(i, carry):
                base = s * RPS + i * G
                loads = [
                    pltpu.async_copy(edg_h.at[base + j], idx[j], sem_i)
                    for j in range(G)
                ]
                for d in loads:
                    d.wait()
                gets = [
                    pltpu.async_copy(table.at[idx[j].at[0]], rows[j], sem_g[j])
                    for j in range(G)
                ]
                puts = []
                for j in range(G):
                    gets[j].wait()
                    puts.append(
                        pltpu.async_copy(
                            rows[j], acc.at[idx[j].at[1]], sem_p, add=True))
                for d in puts:
                    d.wait()
                return carry

            lax.fori_loop(0, RPS // G, body, 0)

        @pl.when(c == 0)
        def _():
            run(h_h.at[0])

        @pl.when(c == 1)
        def _():
            run(h_h.at[1])

        plsc.subcore_barrier()
        sl = pl.ds(s * ROWS_PER_SUB, ROWS_PER_SUB)
        pltpu.sync_copy(acc.at[sl], out_h.at[c, sl])

    return mp_kernel(h, edges3, zerosD)


_BLK = 1024
_GRID = NPAD // _BLK


def _norms(degp_blk):
    d_out = degp_blk[0, 0] + degp_blk[1, 0]
    d_in = degp_blk[0, 1] + degp_blk[1, 1]
    n_src = lax.rsqrt(jnp.where(d_out > 0.0, d_out, 1.0))[:, 0:1]
    n_dst = lax.rsqrt(jnp.where(d_in > 0.0, d_in, 1.0))[:, 0:1]
    return n_src, n_dst


def _split(h):
    return jnp.stack([h[:, :DH], h[:, DH:]], axis=0)


def _tc_scale(featsp, degp):
    def body(f_ref, dg_ref, o_ref):
        n_src, _ = _norms(dg_ref[...])
        o_ref[...] = _split(f_ref[...] * n_src)

    return pl.pallas_call(
        body,
        grid=(_GRID,),
        in_specs=[
            pl.BlockSpec((_BLK, D), lambda i: (i, 0)),
            pl.BlockSpec((NC, 2, _BLK, 8), lambda i: (0, 0, i, 0)),
        ],
        out_specs=pl.BlockSpec((2, _BLK, DH), lambda i: (0, i, 0)),
        out_shape=jax.ShapeDtypeStruct((2, NPAD, DH), jnp.float32),
    )(featsp, degp)


def _merge_parts(p):
    return jnp.concatenate([p[0], p[1]], axis=1)


def _tc_layer(part, degp, W, b):
    def body(p_ref, dg_ref, w_ref, b_ref, o_ref):
        i = pl.program_id(0)
        n_src, n_dst = _norms(dg_ref[...])
        agg = _merge_parts(p_ref[...]) * n_dst
        h = jnp.dot(agg, w_ref[...], preferred_element_type=jnp.float32)
        h = jnp.maximum(h + b_ref[...], 0.0)
        rows = i * _BLK + lax.broadcasted_iota(jnp.int32, (_BLK, 1), 0)
        h = jnp.where(rows < N_NODES, h, 0.0)
        o_ref[...] = _split(h * n_src)

    return pl.pallas_call(
        body,
        grid=(_GRID,),
        in_specs=[
            pl.BlockSpec((2, _BLK, DH), lambda i: (0, i, 0)),
            pl.BlockSpec((NC, 2, _BLK, 8), lambda i: (0, 0, i, 0)),
            pl.BlockSpec((D, D), lambda i: (0, 0)),
            pl.BlockSpec((1, D), lambda i: (0, 0)),
        ],
        out_specs=pl.BlockSpec((2, _BLK, DH), lambda i: (0, i, 0)),
        out_shape=jax.ShapeDtypeStruct((2, NPAD, DH), jnp.float32),
    )(part, degp, W, b)


def _tc_final(part, degp, W, b, Wf1, bf1, Wf2, bf2):
    def body(p_ref, dg_ref, w_ref, b_ref, w1_ref, b1_ref, w2_ref, b2_ref,
             o_ref, acc_ref):
        i = pl.program_id(0)
        _, n_dst = _norms(dg_ref[...])
        agg = _merge_parts(p_ref[...]) * n_dst
        h = jnp.dot(agg, w_ref[...], preferred_element_type=jnp.float32)
        h = jnp.maximum(h + b_ref[...], 0.0)
        rows = i * _BLK + lax.broadcasted_iota(jnp.int32, (_BLK, 1), 0)
        h = jnp.where(rows < N_NODES, h, 0.0)

        @pl.when(i == 0)
        def _():
            acc_ref[...] = jnp.zeros_like(acc_ref)
            o_ref[...] = jnp.zeros_like(o_ref)

        acc_ref[...] += jnp.sum(h, axis=0, keepdims=True)

        @pl.when(i == _GRID - 1)
        def _():
            g = acc_ref[...] * (1.0 / N_NODES)
            x = jnp.dot(g, w1_ref[...], preferred_element_type=jnp.float32)
            x = jnp.maximum(x + b1_ref[...], 0.0)
            l = jnp.dot(x, w2_ref[...], preferred_element_type=jnp.float32)
            l = l + b2_ref[...]
            m = jnp.max(l, axis=1, keepdims=True)
            e = jnp.exp(l - m)
            o_ref[...] = e / jnp.sum(e, axis=1, keepdims=True)

    return pl.pallas_call(
        body,
        grid=(_GRID,),
        in_specs=[
            pl.BlockSpec((2, _BLK, DH), lambda i: (0, i, 0)),
            pl.BlockSpec((NC, 2, _BLK, 8), lambda i: (0, 0, i, 0)),
            pl.BlockSpec((D, D), lambda i: (0, 0)),
            pl.BlockSpec((1, D), lambda i: (0, 0)),
            pl.BlockSpec((D, N_CLASS), lambda i: (0, 0)),
            pl.BlockSpec((1, N_CLASS), lambda i: (0, 0)),
            pl.BlockSpec((N_CLASS, N_CLASS), lambda i: (0, 0)),
            pl.BlockSpec((1, N_CLASS), lambda i: (0, 0)),
        ],
        out_specs=pl.BlockSpec((1, N_CLASS), lambda i: (0, 0)),
        out_shape=jax.ShapeDtypeStruct((1, N_CLASS), jnp.float32),
        scratch_shapes=[pltpu.VMEM((1, D), jnp.float32)],
    )(part, degp, W, b, Wf1, bf1, Wf2, bf2)


def kernel(feats, edge_index, W1, b1, W2, b2, Wf1, bf1, Wf2, bf2):
    src = edge_index[0].astype(jnp.int32)
    dst = edge_index[1].astype(jnp.int32)
    pad = N_NODES + jnp.arange(EPAD - N_EDGES, dtype=jnp.int32) % (
        NPAD - N_NODES)
    src2 = jnp.concatenate([src, pad]).reshape(EROWS, CHUNK)
    dst2 = jnp.concatenate([dst, pad]).reshape(EROWS, CHUNK)
    edges3 = jnp.stack([src2, dst2], axis=1)
    featsp = jnp.pad(feats, ((0, NPAD - N_NODES), (0, 0)))

    ones8 = jnp.ones((CHUNK, 8), jnp.float32)
    zeros8 = jnp.zeros((ROWS_PER_SUB, 8), jnp.float32)
    zerosD = jnp.zeros((CHUNK, DH), jnp.float32)

    degp = _sc_degrees(edges3, ones8, zeros8)
    h0s = _tc_scale(featsp, degp)
    p1 = _sc_scatter(h0s, edges3, zerosD)
    h1s = _tc_layer(p1, degp, W1, b1.reshape(1, D))
    p2 = _sc_scatter(h1s, edges3, zerosD)
    return _tc_final(p2, degp, W2, b2.reshape(1, D),
                     Wf1, bf1.reshape(1, N_CLASS), Wf2,
                     bf2.reshape(1, N_CLASS))


# speedup vs baseline: 1.0543x; 1.0527x over previous
import functools

import jax
import jax.numpy as jnp
from jax import lax
from jax.experimental import pallas as pl
from jax.experimental.pallas import tpu as pltpu
from jax.experimental.pallas import tpu_sc as plsc

N_NODES = 10000
N_EDGES = 320000
D = 128
N_CLASS = 10

NC = 2
NS = 16
NW = NC * NS

NPAD = 10240
ROWS_PER_SUB = NPAD // NS
CHUNK = 128
CH_PER_W = 80
EROWS = NW * CH_PER_W
EPAD = EROWS * CHUNK
DH = D // 2
G = 8

_MESH = dict(core_axis_name="c", subcore_axis_name="s")


def _sc_degrees(edges3, ones8, zeros8):
    @functools.partial(
        pl.kernel,
        out_type=jax.ShapeDtypeStruct((NC, 2, NPAD, 8), jnp.float32),
        mesh=plsc.VectorSubcoreMesh(**_MESH),
        compiler_params=pltpu.CompilerParams(use_tc_tiling_on_sc=False),
        scratch_types=(
            [pltpu.VMEM((2, CHUNK), jnp.int32) for _ in range(G)]
            + [
                pltpu.VMEM((CHUNK, 8), jnp.float32),
                pltpu.VMEM((ROWS_PER_SUB, 8), jnp.float32),
                pltpu.VMEM_SHARED((NPAD, 8), jnp.float32),
                pltpu.VMEM_SHARED((NPAD, 8), jnp.float32),
                pltpu.SemaphoreType.DMA,
                pltpu.SemaphoreType.DMA,
            ]
        ),
    )
    def deg_kernel(edg_h, ones_h, zer_h, out_h, *scr):
        idx = scr[0:G]
        ones_v, zer_v, acc_s, acc_d, sem_i, sem_p = scr[G:]
        c = lax.axis_index("c")
        s = lax.axis_index("s")
        wid = s * NC + c
        pltpu.sync_copy(zer_h, zer_v)
        pltpu.sync_copy(zer_v, acc_s.at[pl.ds(s * ROWS_PER_SUB, ROWS_PER_SUB)])
        pltpu.sync_copy(zer_v, acc_d.at[pl.ds(s * ROWS_PER_SUB, ROWS_PER_SUB)])
        pltpu.sync_copy(ones_h, ones_v)
        plsc.subcore_barrier()

        def body(i, carry):
            base = wid * CH_PER_W + i * G
            loads = [
                pltpu.async_copy(edg_h.at[base + j], idx[j], sem_i)
                for j in range(G)
            ]
            for d in loads:
                d.wait()
            puts = []
            for j in range(G):
                puts.append(
                    pltpu.async_copy(ones_v, acc_s.at[idx[j].at[0]], sem_p,
                                     add=True))
                puts.append(
                    pltpu.async_copy(ones_v, acc_d.at[idx[j].at[1]], sem_p,
                                     add=True))
            for d in puts:
                d.wait()
            return carry

        lax.fori_loop(0, CH_PER_W // G, body, 0)
        plsc.subcore_barrier()
        sl = pl.ds(s * ROWS_PER_SUB, ROWS_PER_SUB)
        pltpu.sync_copy(acc_s.at[sl], out_h.at[c, 0, sl])
        pltpu.sync_copy(acc_d.at[sl], out_h.at[c, 1, sl])

    return deg_kernel(edges3, ones8, zeros8)


RPS = EROWS // NS


def _sc_scatter(h, edges3, zerosD):
    @functools.partial(
        pl.kernel,
        out_type=jax.ShapeDtypeStruct((NPAD, D), jnp.float32),
        mesh=plsc.VectorSubcoreMesh(**_MESH),
        compiler_params=pltpu.CompilerParams(use_tc_tiling_on_sc=False),
        scratch_types=(
            [pltpu.VMEM((2, CHUNK), jnp.int32) for _ in range(G)]
            + [pltpu.VMEM((CHUNK, DH), jnp.float32) for _ in range(G)]
            + [
                pltpu.VMEM((CHUNK, DH), jnp.float32),
                pltpu.VMEM_SHARED((NPAD, DH), jnp.float32),
                pltpu.SemaphoreType.DMA,
                pltpu.SemaphoreType.DMA,
            ]
            + [pltpu.SemaphoreType.DMA for _ in range(G)]
        ),
    )
    def mp_kernel(h_h, edg_h, zer_h, out_h, *scr):
        idx = scr[0:G]
        rows = scr[G:2 * G]
        zer_v = scr[2 * G]
        acc = scr[2 * G + 1]
        sem_i = scr[2 * G + 2]
        sem_p = scr[2 * G + 3]
        sem_g = scr[2 * G + 4:2 * G + 4 + G]
        c = lax.axis_index("c")
        s = lax.axis_index("s")
        pltpu.sync_copy(zer_h, zer_v)
        for k in range(ROWS_PER_SUB // CHUNK):
            pltpu.sync_copy(
                zer_v, acc.at[pl.ds(s * ROWS_PER_SUB + k * CHUNK, CHUNK)])
        plsc.subcore_barrier()

        def run(table):
            def body(i, carry):
                base = s * RPS + i * G
                loads = [
                    pltpu.async_copy(edg_h.at[base + j], idx[j], sem_i)
                    for j in range(G)
                ]
                for d in loads:
                    d.wait()
                gets = [
                    pltpu.async_copy(table.at[idx[j].at[0]], rows[j], sem_g[j])
                    for j in range(G)
                ]
                puts = []
                for j in range(G):
                    gets[j].wait()
                    puts.append(
                        pltpu.async_copy(
                            rows[j], acc.at[idx[j].at[1]], sem_p, add=True))
                for d in puts:
                    d.wait()
                return carry

            lax.fori_loop(0, RPS // G, body, 0)

        @pl.when(c == 0)
        def _():
            run(h_h.at[0])

        @pl.when(c == 1)
        def _():
            run(h_h.at[1])

        plsc.subcore_barrier()
        sl = pl.ds(s * ROWS_PER_SUB, ROWS_PER_SUB)
        pltpu.sync_copy(acc.at[sl], out_h.at[sl, pl.ds(c * DH, DH)])

    return mp_kernel(h, edges3, zerosD)


_BLK = 1024
_GRID = NPAD // _BLK


def _norms(degp_blk):
    d_out = degp_blk[0, 0] + degp_blk[1, 0]
    d_in = degp_blk[0, 1] + degp_blk[1, 1]
    n_src = lax.rsqrt(jnp.where(d_out > 0.0, d_out, 1.0))[:, 0:1]
    n_dst = lax.rsqrt(jnp.where(d_in > 0.0, d_in, 1.0))[:, 0:1]
    return n_src, n_dst


def _split(h):
    return jnp.stack([h[:, :DH], h[:, DH:]], axis=0)


def _tc_scale(featsp, degp):
    def body(f_ref, dg_ref, o_ref):
        n_src, _ = _norms(dg_ref[...])
        o_ref[...] = _split(f_ref[...] * n_src)

    return pl.pallas_call(
        body,
        grid=(_GRID,),
        in_specs=[
            pl.BlockSpec((_BLK, D), lambda i: (i, 0)),
            pl.BlockSpec((NC, 2, _BLK, 8), lambda i: (0, 0, i, 0)),
        ],
        out_specs=pl.BlockSpec((2, _BLK, DH), lambda i: (0, i, 0)),
        out_shape=jax.ShapeDtypeStruct((2, NPAD, DH), jnp.float32),
    )(featsp, degp)


def _tc_layer(part, degp, W, b):
    def body(p_ref, dg_ref, w_ref, b_ref, o_ref):
        i = pl.program_id(0)
        n_src, n_dst = _norms(dg_ref[...])
        agg = p_ref[...] * n_dst
        h = jnp.dot(agg, w_ref[...], preferred_element_type=jnp.float32)
        h = jnp.maximum(h + b_ref[...], 0.0)
        rows = i * _BLK + lax.broadcasted_iota(jnp.int32, (_BLK, 1), 0)
        h = jnp.where(rows < N_NODES, h, 0.0)
        o_ref[...] = _split(h * n_src)

    return pl.pallas_call(
        body,
        grid=(_GRID,),
        in_specs=[
            pl.BlockSpec((_BLK, D), lambda i: (i, 0)),
            pl.BlockSpec((NC, 2, _BLK, 8), lambda i: (0, 0, i, 0)),
            pl.BlockSpec((D, D), lambda i: (0, 0)),
            pl.BlockSpec((1, D), lambda i: (0, 0)),
        ],
        out_specs=pl.BlockSpec((2, _BLK, DH), lambda i: (0, i, 0)),
        out_shape=jax.ShapeDtypeStruct((2, NPAD, DH), jnp.float32),
    )(part, degp, W, b)


def _tc_final(part, degp, W, b, Wf1, bf1, Wf2, bf2):
    def body(p_ref, dg_ref, w_ref, b_ref, w1_ref, b1_ref, w2_ref, b2_ref,
             o_ref, acc_ref):
        i = pl.program_id(0)
        _, n_dst = _norms(dg_ref[...])
        agg = p_ref[...] * n_dst
        h = jnp.dot(agg, w_ref[...], preferred_element_type=jnp.float32)
        h = jnp.maximum(h + b_ref[...], 0.0)
        rows = i * _BLK + lax.broadcasted_iota(jnp.int32, (_BLK, 1), 0)
        h = jnp.where(rows < N_NODES, h, 0.0)

        @pl.when(i == 0)
        def _():
            acc_ref[...] = jnp.zeros_like(acc_ref)
            o_ref[...] = jnp.zeros_like(o_ref)

        acc_ref[...] += jnp.sum(h, axis=0, keepdims=True)

        @pl.when(i == _GRID - 1)
        def _():
            g = acc_ref[...] * (1.0 / N_NODES)
            x = jnp.dot(g, w1_ref[...], preferred_element_type=jnp.float32)
            x = jnp.maximum(x + b1_ref[...], 0.0)
            l = jnp.dot(x, w2_ref[...], preferred_element_type=jnp.float32)
            l = l + b2_ref[...]
            m = jnp.max(l, axis=1, keepdims=True)
            e = jnp.exp(l - m)
            o_ref[...] = e / jnp.sum(e, axis=1, keepdims=True)

    return pl.pallas_call(
        body,
        grid=(_GRID,),
        in_specs=[
            pl.BlockSpec((_BLK, D), lambda i: (i, 0)),
            pl.BlockSpec((NC, 2, _BLK, 8), lambda i: (0, 0, i, 0)),
            pl.BlockSpec((D, D), lambda i: (0, 0)),
            pl.BlockSpec((1, D), lambda i: (0, 0)),
            pl.BlockSpec((D, N_CLASS), lambda i: (0, 0)),
            pl.BlockSpec((1, N_CLASS), lambda i: (0, 0)),
            pl.BlockSpec((N_CLASS, N_CLASS), lambda i: (0, 0)),
            pl.BlockSpec((1, N_CLASS), lambda i: (0, 0)),
        ],
        out_specs=pl.BlockSpec((1, N_CLASS), lambda i: (0, 0)),
        out_shape=jax.ShapeDtypeStruct((1, N_CLASS), jnp.float32),
        scratch_shapes=[pltpu.VMEM((1, D), jnp.float32)],
    )(part, degp, W, b, Wf1, bf1, Wf2, bf2)


def kernel(feats, edge_index, W1, b1, W2, b2, Wf1, bf1, Wf2, bf2):
    src = edge_index[0].astype(jnp.int32)
    dst = edge_index[1].astype(jnp.int32)
    pad = N_NODES + jnp.arange(EPAD - N_EDGES, dtype=jnp.int32) % (
        NPAD - N_NODES)
    src2 = jnp.concatenate([src, pad]).reshape(EROWS, CHUNK)
    dst2 = jnp.concatenate([dst, pad]).reshape(EROWS, CHUNK)
    edges3 = jnp.stack([src2, dst2], axis=1)
    featsp = jnp.pad(feats, ((0, NPAD - N_NODES), (0, 0)))

    ones8 = jnp.ones((CHUNK, 8), jnp.float32)
    zeros8 = jnp.zeros((ROWS_PER_SUB, 8), jnp.float32)
    zerosD = jnp.zeros((CHUNK, DH), jnp.float32)

    degp = _sc_degrees(edges3, ones8, zeros8)
    h0s = _tc_scale(featsp, degp)
    p1 = _sc_scatter(h0s, edges3, zerosD)
    h1s = _tc_layer(p1, degp, W1, b1.reshape(1, D))
    p2 = _sc_scatter(h1s, edges3, zerosD)
    return _tc_final(p2, degp, W2, b2.reshape(1, D),
                     Wf1, bf1.reshape(1, N_CLASS), Wf2,
                     bf2.reshape(1, N_CLASS))


# speedup vs baseline: 1.1058x; 1.0489x over previous
import functools

import jax
import jax.numpy as jnp
from jax import lax
from jax.experimental import pallas as pl
from jax.experimental.pallas import tpu as pltpu
from jax.experimental.pallas import tpu_sc as plsc

N_NODES = 10000
N_EDGES = 320000
D = 128
N_CLASS = 10

NC = 2
NS = 16
NW = NC * NS

NPAD = 10240
ROWS_PER_SUB = NPAD // NS
CHUNK = 128
CH_PER_W = 80
EROWS = NW * CH_PER_W
EPAD = EROWS * CHUNK
DH = D // 2
G = 8

_MESH = dict(core_axis_name="c", subcore_axis_name="s")


def _sc_degrees(edges3, ones8, zeros8):
    @functools.partial(
        pl.kernel,
        out_type=jax.ShapeDtypeStruct((NC, 2, NPAD, 8), jnp.float32),
        mesh=plsc.VectorSubcoreMesh(**_MESH),
        compiler_params=pltpu.CompilerParams(use_tc_tiling_on_sc=False),
        scratch_types=(
            [pltpu.VMEM((2, CHUNK), jnp.int32) for _ in range(G)]
            + [
                pltpu.VMEM((CHUNK, 8), jnp.float32),
                pltpu.VMEM((ROWS_PER_SUB, 8), jnp.float32),
                pltpu.VMEM_SHARED((NPAD, 8), jnp.float32),
                pltpu.VMEM_SHARED((NPAD, 8), jnp.float32),
                pltpu.SemaphoreType.DMA,
                pltpu.SemaphoreType.DMA,
            ]
        ),
    )
    def deg_kernel(edg_h, ones_h, zer_h, out_h, *scr):
        idx = scr[0:G]
        ones_v, zer_v, acc_s, acc_d, sem_i, sem_p = scr[G:]
        c = lax.axis_index("c")
        s = lax.axis_index("s")
        wid = s * NC + c
        pltpu.sync_copy(zer_h, zer_v)
        pltpu.sync_copy(zer_v, acc_s.at[pl.ds(s * ROWS_PER_SUB, ROWS_PER_SUB)])
        pltpu.sync_copy(zer_v, acc_d.at[pl.ds(s * ROWS_PER_SUB, ROWS_PER_SUB)])
        pltpu.sync_copy(ones_h, ones_v)
        plsc.subcore_barrier()

        def body(i, carry):
            base = wid * CH_PER_W + i * G
            loads = [
                pltpu.async_copy(edg_h.at[base + j], idx[j], sem_i)
                for j in range(G)
            ]
            for d in loads:
                d.wait()
            puts = []
            for j in range(G):
                puts.append(
                    pltpu.async_copy(ones_v, acc_s.at[idx[j].at[0]], sem_p,
                                     add=True))
                puts.append(
                    pltpu.async_copy(ones_v, acc_d.at[idx[j].at[1]], sem_p,
                                     add=True))
            for d in puts:
                d.wait()
            return carry

        lax.fori_loop(0, CH_PER_W // G, body, 0)
        plsc.subcore_barrier()
        sl = pl.ds(s * ROWS_PER_SUB, ROWS_PER_SUB)
        pltpu.sync_copy(acc_s.at[sl], out_h.at[c, 0, sl])
        pltpu.sync_copy(acc_d.at[sl], out_h.at[c, 1, sl])

    return deg_kernel(edges3, ones8, zeros8)


RPS = EROWS // NS


def _sc_scatter(h, edges3, zerosD):
    @functools.partial(
        pl.kernel,
        out_type=jax.ShapeDtypeStruct((NPAD, D), jnp.float32),
        mesh=plsc.VectorSubcoreMesh(**_MESH),
        compiler_params=pltpu.CompilerParams(use_tc_tiling_on_sc=False),
        scratch_types=(
            [pltpu.VMEM((2, CHUNK), jnp.int32) for _ in range(G)]
            + [pltpu.VMEM((CHUNK,), jnp.int32) for _ in range(G)]
            + [pltpu.VMEM((CHUNK, DH), jnp.float32) for _ in range(G)]
            + [
                pltpu.VMEM((CHUNK, DH), jnp.float32),
                pltpu.VMEM_SHARED((NPAD, DH), jnp.float32),
                pltpu.SemaphoreType.DMA,
                pltpu.SemaphoreType.DMA,
            ]
            + [pltpu.SemaphoreType.DMA for _ in range(G)]
        ),
    )
    def mp_kernel(h_h, edg_h, zer_h, out_h, *scr):
        idx = scr[0:G]
        idx2 = scr[G:2 * G]
        rows = scr[2 * G:3 * G]
        zer_v = scr[3 * G]
        acc = scr[3 * G + 1]
        sem_i = scr[3 * G + 2]
        sem_p = scr[3 * G + 3]
        sem_g = scr[3 * G + 4:3 * G + 4 + G]
        c = lax.axis_index("c")
        s = lax.axis_index("s")
        pltpu.sync_copy(zer_h, zer_v)
        for k in range(ROWS_PER_SUB // CHUNK):
            pltpu.sync_copy(
                zer_v, acc.at[pl.ds(s * ROWS_PER_SUB + k * CHUNK, CHUNK)])
        plsc.subcore_barrier()

        def body(i, carry):
            base = s * RPS + i * G
            loads = [
                pltpu.async_copy(edg_h.at[base + j], idx[j], sem_i)
                for j in range(G)
            ]
            for d in loads:
                d.wait()
            for j in range(G):
                for k in range(CHUNK // 16):
                    sl16 = pl.ds(k * 16, 16)
                    idx2[j][sl16] = idx[j][0, sl16] * 2 + c
            gets = [
                pltpu.async_copy(h_h.at[idx2[j]], rows[j], sem_g[j])
                for j in range(G)
            ]
            puts = []
            for j in range(G):
                gets[j].wait()
                puts.append(
                    pltpu.async_copy(
                        rows[j], acc.at[idx[j].at[1]], sem_p, add=True))
            for d in puts:
                d.wait()
            return carry

        lax.fori_loop(0, RPS // G, body, 0)
        plsc.subcore_barrier()
        sl = pl.ds(s * ROWS_PER_SUB, ROWS_PER_SUB)
        pltpu.sync_copy(acc.at[sl], out_h.at[sl, pl.ds(c * DH, DH)])

    return mp_kernel(h, edges3, zerosD)


_BLK = 1024
_GRID = NPAD // _BLK


def _norms(degp_blk):
    d_out = degp_blk[0, 0] + degp_blk[1, 0]
    d_in = degp_blk[0, 1] + degp_blk[1, 1]
    n_src = lax.rsqrt(jnp.where(d_out > 0.0, d_out, 1.0))[:, 0:1]
    n_dst = lax.rsqrt(jnp.where(d_in > 0.0, d_in, 1.0))[:, 0:1]
    return n_src, n_dst


def _tc_scale(featsp, degp):
    def body(f_ref, dg_ref, o_ref):
        n_src, _ = _norms(dg_ref[...])
        o_ref[...] = f_ref[...] * n_src

    return pl.pallas_call(
        body,
        grid=(_GRID,),
        in_specs=[
            pl.BlockSpec((_BLK, D), lambda i: (i, 0)),
            pl.BlockSpec((NC, 2, _BLK, 8), lambda i: (0, 0, i, 0)),
        ],
        out_specs=pl.BlockSpec((_BLK, D), lambda i: (i, 0)),
        out_shape=jax.ShapeDtypeStruct((NPAD, D), jnp.float32),
    )(featsp, degp)


def _tc_layer(part, degp, W, b):
    def body(p_ref, dg_ref, w_ref, b_ref, o_ref):
        i = pl.program_id(0)
        n_src, n_dst = _norms(dg_ref[...])
        agg = p_ref[...] * n_dst
        h = jnp.dot(agg, w_ref[...], preferred_element_type=jnp.float32)
        h = jnp.maximum(h + b_ref[...], 0.0)
        rows = i * _BLK + lax.broadcasted_iota(jnp.int32, (_BLK, 1), 0)
        h = jnp.where(rows < N_NODES, h, 0.0)
        o_ref[...] = h * n_src

    return pl.pallas_call(
        body,
        grid=(_GRID,),
        in_specs=[
            pl.BlockSpec((_BLK, D), lambda i: (i, 0)),
            pl.BlockSpec((NC, 2, _BLK, 8), lambda i: (0, 0, i, 0)),
            pl.BlockSpec((D, D), lambda i: (0, 0)),
            pl.BlockSpec((1, D), lambda i: (0, 0)),
        ],
        out_specs=pl.BlockSpec((_BLK, D), lambda i: (i, 0)),
        out_shape=jax.ShapeDtypeStruct((NPAD, D), jnp.float32),
    )(part, degp, W, b)


def _tc_final(part, degp, W, b, Wf1, bf1, Wf2, bf2):
    def body(p_ref, dg_ref, w_ref, b_ref, w1_ref, b1_ref, w2_ref, b2_ref,
             o_ref, acc_ref):
        i = pl.program_id(0)
        _, n_dst = _norms(dg_ref[...])
        agg = p_ref[...] * n_dst
        h = jnp.dot(agg, w_ref[...], preferred_element_type=jnp.float32)
        h = jnp.maximum(h + b_ref[...], 0.0)
        rows = i * _BLK + lax.broadcasted_iota(jnp.int32, (_BLK, 1), 0)
        h = jnp.where(rows < N_NODES, h, 0.0)

        @pl.when(i == 0)
        def _():
            acc_ref[...] = jnp.zeros_like(acc_ref)
            o_ref[...] = jnp.zeros_like(o_ref)

        acc_ref[...] += jnp.sum(h, axis=0, keepdims=True)

        @pl.when(i == _GRID - 1)
        def _():
            g = acc_ref[...] * (1.0 / N_NODES)
            x = jnp.dot(g, w1_ref[...], preferred_element_type=jnp.float32)
            x = jnp.maximum(x + b1_ref[...], 0.0)
            l = jnp.dot(x, w2_ref[...], preferred_element_type=jnp.float32)
            l = l + b2_ref[...]
            m = jnp.max(l, axis=1, keepdims=True)
            e = jnp.exp(l - m)
            o_ref[...] = e / jnp.sum(e, axis=1, keepdims=True)

    return pl.pallas_call(
        body,
        grid=(_GRID,),
        in_specs=[
            pl.BlockSpec((_BLK, D), lambda i: (i, 0)),
            pl.BlockSpec((NC, 2, _BLK, 8), lambda i: (0, 0, i, 0)),
            pl.BlockSpec((D, D), lambda i: (0, 0)),
            pl.BlockSpec((1, D), lambda i: (0, 0)),
            pl.BlockSpec((D, N_CLASS), lambda i: (0, 0)),
            pl.BlockSpec((1, N_CLASS), lambda i: (0, 0)),
            pl.BlockSpec((N_CLASS, N_CLASS), lambda i: (0, 0)),
            pl.BlockSpec((1, N_CLASS), lambda i: (0, 0)),
        ],
        out_specs=pl.BlockSpec((1, N_CLASS), lambda i: (0, 0)),
        out_shape=jax.ShapeDtypeStruct((1, N_CLASS), jnp.float32),
        scratch_shapes=[pltpu.VMEM((1, D), jnp.float32)],
    )(part, degp, W, b, Wf1, bf1, Wf2, bf2)


def kernel(feats, edge_index, W1, b1, W2, b2, Wf1, bf1, Wf2, bf2):
    src = edge_index[0].astype(jnp.int32)
    dst = edge_index[1].astype(jnp.int32)
    pad = N_NODES + jnp.arange(EPAD - N_EDGES, dtype=jnp.int32) % (
        NPAD - N_NODES)
    src2 = jnp.concatenate([src, pad]).reshape(EROWS, CHUNK)
    dst2 = jnp.concatenate([dst, pad]).reshape(EROWS, CHUNK)
    edges3 = jnp.stack([src2, dst2], axis=1)
    featsp = jnp.pad(feats, ((0, NPAD - N_NODES), (0, 0)))

    ones8 = jnp.ones((CHUNK, 8), jnp.float32)
    zeros8 = jnp.zeros((ROWS_PER_SUB, 8), jnp.float32)
    zerosD = jnp.zeros((CHUNK, DH), jnp.float32)

    degp = _sc_degrees(edges3, ones8, zeros8)
    h0s = _tc_scale(featsp, degp)
    p1 = _sc_scatter(h0s.reshape(2 * NPAD, DH), edges3, zerosD)
    h1s = _tc_layer(p1, degp, W1, b1.reshape(1, D))
    p2 = _sc_scatter(h1s.reshape(2 * NPAD, DH), edges3, zerosD)
    return _tc_final(p2, degp, W2, b2.reshape(1, D),
                     Wf1, bf1.reshape(1, N_CLASS), Wf2,
                     bf2.reshape(1, N_CLASS))


# speedup vs baseline: 1.1070x; 1.0011x over previous
import functools

import jax
import jax.numpy as jnp
from jax import lax
from jax.experimental import pallas as pl
from jax.experimental.pallas import tpu as pltpu
from jax.experimental.pallas import tpu_sc as plsc

N_NODES = 10000
N_EDGES = 320000
D = 128
N_CLASS = 10

NC = 2
NS = 16
NW = NC * NS

NPAD = 10240
ROWS_PER_SUB = NPAD // NS
CHUNK = 128
CH_PER_W = 80
EROWS = NW * CH_PER_W
EPAD = EROWS * CHUNK
DH = D // 2
G = 8

_MESH = dict(core_axis_name="c", subcore_axis_name="s")


def _sc_degrees(edges3, ones8, zeros8):
    @functools.partial(
        pl.kernel,
        out_type=jax.ShapeDtypeStruct((NC, 2, NPAD, 8), jnp.float32),
        mesh=plsc.VectorSubcoreMesh(**_MESH),
        compiler_params=pltpu.CompilerParams(use_tc_tiling_on_sc=False),
        scratch_types=(
            [pltpu.VMEM((2, CHUNK), jnp.int32) for _ in range(G)]
            + [
                pltpu.VMEM((CHUNK, 8), jnp.float32),
                pltpu.VMEM((ROWS_PER_SUB, 8), jnp.float32),
                pltpu.VMEM_SHARED((NPAD, 8), jnp.float32),
                pltpu.VMEM_SHARED((NPAD, 8), jnp.float32),
                pltpu.SemaphoreType.DMA,
                pltpu.SemaphoreType.DMA,
            ]
        ),
    )
    def deg_kernel(edg_h, ones_h, zer_h, out_h, *scr):
        idx = scr[0:G]
        ones_v, zer_v, acc_s, acc_d, sem_i, sem_p = scr[G:]
        c = lax.axis_index("c")
        s = lax.axis_index("s")
        wid = s * NC + c
        pltpu.sync_copy(zer_h, zer_v)
        pltpu.sync_copy(zer_v, acc_s.at[pl.ds(s * ROWS_PER_SUB, ROWS_PER_SUB)])
        pltpu.sync_copy(zer_v, acc_d.at[pl.ds(s * ROWS_PER_SUB, ROWS_PER_SUB)])
        pltpu.sync_copy(ones_h, ones_v)
        plsc.subcore_barrier()

        def body(i, carry):
            base = wid * CH_PER_W + i * G
            loads = [
                pltpu.async_copy(edg_h.at[base + j], idx[j], sem_i)
                for j in range(G)
            ]
            for d in loads:
                d.wait()
            puts = []
            for j in range(G):
                puts.append(
                    pltpu.async_copy(ones_v, acc_s.at[idx[j].at[0]], sem_p,
                                     add=True))
                puts.append(
                    pltpu.async_copy(ones_v, acc_d.at[idx[j].at[1]], sem_p,
                                     add=True))
            for d in puts:
                d.wait()
            return carry

        lax.fori_loop(0, CH_PER_W // G, body, 0)
        plsc.subcore_barrier()
        sl = pl.ds(s * ROWS_PER_SUB, ROWS_PER_SUB)
        pltpu.sync_copy(acc_s.at[sl], out_h.at[c, 0, sl])
        pltpu.sync_copy(acc_d.at[sl], out_h.at[c, 1, sl])

    return deg_kernel(edges3, ones8, zeros8)


RPS = EROWS // NS


def _sc_scatter(h, edges3, zerosD):
    @functools.partial(
        pl.kernel,
        out_type=jax.ShapeDtypeStruct((NPAD, D), jnp.float32),
        mesh=plsc.VectorSubcoreMesh(**_MESH),
        compiler_params=pltpu.CompilerParams(use_tc_tiling_on_sc=False),
        scratch_types=(
            [pltpu.VMEM((2, CHUNK), jnp.int32) for _ in range(G)]
            + [pltpu.VMEM((CHUNK,), jnp.int32) for _ in range(G)]
            + [pltpu.VMEM((CHUNK, DH), jnp.float32) for _ in range(G)]
            + [
                pltpu.VMEM((CHUNK, DH), jnp.float32),
                pltpu.VMEM_SHARED((NPAD, DH), jnp.float32),
                pltpu.SemaphoreType.DMA,
                pltpu.SemaphoreType.DMA,
            ]
            + [pltpu.SemaphoreType.DMA for _ in range(G)]
        ),
    )
    def mp_kernel(h_h, edg_h, zer_h, out_h, *scr):
        idx = scr[0:G]
        idx2 = scr[G:2 * G]
        rows = scr[2 * G:3 * G]
        zer_v = scr[3 * G]
        acc = scr[3 * G + 1]
        sem_i = scr[3 * G + 2]
        sem_p = scr[3 * G + 3]
        sem_g = scr[3 * G + 4:3 * G + 4 + G]
        c = lax.axis_index("c")
        s = lax.axis_index("s")
        pltpu.sync_copy(zer_h, zer_v)
        for k in range(ROWS_PER_SUB // CHUNK):
            pltpu.sync_copy(
                zer_v, acc.at[pl.ds(s * ROWS_PER_SUB + k * CHUNK, CHUNK)])
        plsc.subcore_barrier()

        def body(i, carry):
            base = s * RPS + i * G
            loads = [
                pltpu.async_copy(edg_h.at[base + j], idx[j], sem_i)
                for j in range(G)
            ]
            for d in loads:
                d.wait()
            for j in range(G):
                for k in range(CHUNK // 16):
                    sl16 = pl.ds(k * 16, 16)
                    idx2[j][sl16] = idx[j][0, sl16] * 2 + c
            gets = [
                pltpu.async_copy(h_h.at[idx2[j]], rows[j], sem_g[j])
                for j in range(G)
            ]
            puts = []
            for j in range(G):
                gets[j].wait()
                puts.append(
                    pltpu.async_copy(
                        rows[j], acc.at[idx[j].at[1]], sem_p, add=True))
            for d in puts:
                d.wait()
            return carry

        lax.fori_loop(0, RPS // G, body, 0)
        plsc.subcore_barrier()
        sl = pl.ds(s * ROWS_PER_SUB, ROWS_PER_SUB)
        pltpu.sync_copy(acc.at[sl], out_h.at[sl, pl.ds(c * DH, DH)])

    return mp_kernel(h, edges3, zerosD)


_BLK = 1024
_GRID = NPAD // _BLK


def _norms(degp_blk):
    d_out = degp_blk[0, 0] + degp_blk[1, 0]
    d_in = degp_blk[0, 1] + degp_blk[1, 1]
    n_src = lax.rsqrt(jnp.where(d_out > 0.0, d_out, 1.0))[:, 0:1]
    n_dst = lax.rsqrt(jnp.where(d_in > 0.0, d_in, 1.0))[:, 0:1]
    return n_src, n_dst


def _tc_scale(featsp, degp):
    def body(f_ref, dg_ref, o_ref, ns_ref, nd_ref):
        n_src, n_dst = _norms(dg_ref[...])
        o_ref[...] = f_ref[...] * n_src
        ns_ref[...] = jnp.broadcast_to(n_src, (_BLK, D))
        nd_ref[...] = jnp.broadcast_to(n_dst, (_BLK, D))

    return pl.pallas_call(
        body,
        grid=(_GRID,),
        in_specs=[
            pl.BlockSpec((_BLK, D), lambda i: (i, 0)),
            pl.BlockSpec((NC, 2, _BLK, 8), lambda i: (0, 0, i, 0)),
        ],
        out_specs=[pl.BlockSpec((_BLK, D), lambda i: (i, 0))] * 3,
        out_shape=[jax.ShapeDtypeStruct((NPAD, D), jnp.float32)] * 3,
    )(featsp, degp)


def _tc_layer(part, nsrc, ndst, W, b):
    def body(p_ref, ns_ref, nd_ref, w_ref, b_ref, o_ref):
        i = pl.program_id(0)
        agg = p_ref[...] * nd_ref[...]
        h = jnp.dot(agg, w_ref[...], preferred_element_type=jnp.float32)
        h = jnp.maximum(h + b_ref[...], 0.0)
        rows = i * _BLK + lax.broadcasted_iota(jnp.int32, (_BLK, 1), 0)
        h = jnp.where(rows < N_NODES, h, 0.0)
        o_ref[...] = h * ns_ref[...]

    return pl.pallas_call(
        body,
        grid=(_GRID,),
        in_specs=[
            pl.BlockSpec((_BLK, D), lambda i: (i, 0)),
            pl.BlockSpec((_BLK, D), lambda i: (i, 0)),
            pl.BlockSpec((_BLK, D), lambda i: (i, 0)),
            pl.BlockSpec((D, D), lambda i: (0, 0)),
            pl.BlockSpec((1, D), lambda i: (0, 0)),
        ],
        out_specs=pl.BlockSpec((_BLK, D), lambda i: (i, 0)),
        out_shape=jax.ShapeDtypeStruct((NPAD, D), jnp.float32),
    )(part, nsrc, ndst, W, b)


def _tc_final(part, ndst, W, b, Wf1, bf1, Wf2, bf2):
    def body(p_ref, nd_ref, w_ref, b_ref, w1_ref, b1_ref, w2_ref, b2_ref,
             o_ref, acc_ref):
        i = pl.program_id(0)
        agg = p_ref[...] * nd_ref[...]
        h = jnp.dot(agg, w_ref[...], preferred_element_type=jnp.float32)
        h = jnp.maximum(h + b_ref[...], 0.0)
        rows = i * _BLK + lax.broadcasted_iota(jnp.int32, (_BLK, 1), 0)
        h = jnp.where(rows < N_NODES, h, 0.0)

        @pl.when(i == 0)
        def _():
            acc_ref[...] = jnp.zeros_like(acc_ref)
            o_ref[...] = jnp.zeros_like(o_ref)

        acc_ref[...] += jnp.sum(h, axis=0, keepdims=True)

        @pl.when(i == _GRID - 1)
        def _():
            g = acc_ref[...] * (1.0 / N_NODES)
            x = jnp.dot(g, w1_ref[...], preferred_element_type=jnp.float32)
            x = jnp.maximum(x + b1_ref[...], 0.0)
            l = jnp.dot(x, w2_ref[...], preferred_element_type=jnp.float32)
            l = l + b2_ref[...]
            m = jnp.max(l, axis=1, keepdims=True)
            e = jnp.exp(l - m)
            o_ref[...] = e / jnp.sum(e, axis=1, keepdims=True)

    return pl.pallas_call(
        body,
        grid=(_GRID,),
        in_specs=[
            pl.BlockSpec((_BLK, D), lambda i: (i, 0)),
            pl.BlockSpec((_BLK, D), lambda i: (i, 0)),
            pl.BlockSpec((D, D), lambda i: (0, 0)),
            pl.BlockSpec((1, D), lambda i: (0, 0)),
            pl.BlockSpec((D, N_CLASS), lambda i: (0, 0)),
            pl.BlockSpec((1, N_CLASS), lambda i: (0, 0)),
            pl.BlockSpec((N_CLASS, N_CLASS), lambda i: (0, 0)),
            pl.BlockSpec((1, N_CLASS), lambda i: (0, 0)),
        ],
        out_specs=pl.BlockSpec((1, N_CLASS), lambda i: (0, 0)),
        out_shape=jax.ShapeDtypeStruct((1, N_CLASS), jnp.float32),
        scratch_shapes=[pltpu.VMEM((1, D), jnp.float32)],
    )(part, ndst, W, b, Wf1, bf1, Wf2, bf2)


def kernel(feats, edge_index, W1, b1, W2, b2, Wf1, bf1, Wf2, bf2):
    src = edge_index[0].astype(jnp.int32)
    dst = edge_index[1].astype(jnp.int32)
    pad = N_NODES + jnp.arange(EPAD - N_EDGES, dtype=jnp.int32) % (
        NPAD - N_NODES)
    src2 = jnp.concatenate([src, pad]).reshape(EROWS, CHUNK)
    dst2 = jnp.concatenate([dst, pad]).reshape(EROWS, CHUNK)
    edges3 = jnp.stack([src2, dst2], axis=1)
    featsp = jnp.pad(feats, ((0, NPAD - N_NODES), (0, 0)))

    ones8 = jnp.ones((CHUNK, 8), jnp.float32)
    zeros8 = jnp.zeros((ROWS_PER_SUB, 8), jnp.float32)
    zerosD = jnp.zeros((CHUNK, DH), jnp.float32)

    degp = _sc_degrees(edges3, ones8, zeros8)
    h0s, nsrc, ndst = _tc_scale(featsp, degp)
    p1 = _sc_scatter(h0s.reshape(2 * NPAD, DH), edges3, zerosD)
    h1s = _tc_layer(p1, nsrc, ndst, W1, b1.reshape(1, D))
    p2 = _sc_scatter(h1s.reshape(2 * NPAD, DH), edges3, zerosD)
    return _tc_final(p2, ndst, W2, b2.reshape(1, D),
                     Wf1, bf1.reshape(1, N_CLASS), Wf2,
                     bf2.reshape(1, N_CLASS))
